# Initial kernel scaffold; baseline (speedup 1.0000x reference)
import jax
import jax.numpy as jnp
from jax.experimental import pallas as pl


def kernel(h, edge_index, W1, b1, W2, b2, g1, be1, g2, be2, g3, be3):
    raise NotImplementedError("write your pallas kernel here")


# speedup vs baseline: 3.1204x; 3.1204x over previous
import functools

import jax
import jax.numpy as jnp
from jax import lax
from jax.experimental import pallas as pl
from jax.experimental.pallas import tpu as pltpu
from jax.experimental.pallas import tpu_sc as plsc

N_NODES = 10000
N_EDGES = 160000
D = 256
H = 128
NUM_LAYERS = 3
BN_EPS = 1e-5

NC = 2
NS = 16
EPT = N_EDGES // NS
CHUNK = 80
NCHUNK = EPT // CHUNK
NP = 10240
RPT = NP // NS


def _seg_body(h_hbm, src_hbm, dst_hbm, out_hbm, acc_sh, src_v, dst_v, rows_v):
    c = lax.axis_index("c")
    s = lax.axis_index("s")
    pltpu.sync_copy(h_hbm.at[c, pl.ds(s * RPT, RPT)],
                    acc_sh.at[pl.ds(s * RPT, RPT)])
    plsc.subcore_barrier()

    @pl.loop(0, NCHUNK)
    def _(j):
        off = s * EPT + j * CHUNK
        pltpu.sync_copy(src_hbm.at[pl.ds(off, CHUNK)], src_v)
        pltpu.sync_copy(dst_hbm.at[pl.ds(off, CHUNK)], dst_v)
        pltpu.sync_copy(h_hbm.at[c].at[src_v], rows_v)
        pltpu.sync_copy(rows_v, acc_sh.at[dst_v], add=True)

    plsc.subcore_barrier()
    pltpu.sync_copy(acc_sh.at[pl.ds(s * RPT, RPT)],
                    out_hbm.at[c, pl.ds(s * RPT, RPT)])


def _segment_sum(h_split, src, dst):
    mesh = plsc.VectorSubcoreMesh(core_axis_name="c", subcore_axis_name="s")
    return pl.kernel(
        _seg_body,
        out_type=jax.ShapeDtypeStruct((NC, NP, H), jnp.float32),
        mesh=mesh,
        scratch_types=[
            pltpu.VMEM_SHARED((NP, H), jnp.float32),
            pltpu.VMEM((CHUNK,), jnp.int32),
            pltpu.VMEM((CHUNK,), jnp.int32),
            pltpu.VMEM((CHUNK, H), jnp.float32),
        ],
    )(h_split, src, dst)


def _bn_relu(t, g, be):
    mu = jnp.mean(t, axis=0, keepdims=True)
    d = t - mu
    var = jnp.mean(d * d, axis=0, keepdims=True)
    return jnp.maximum(g * d * lax.rsqrt(var + BN_EPS) + be, 0.0)


def _mlp_body(split_out, x_ref, w1_ref, b1_ref, w2_ref, b2_ref,
              g1_ref, be1_ref, g2_ref, be2_ref, g3_ref, be3_ref, out_ref):
    x = jnp.concatenate([x_ref[0, :N_NODES], x_ref[1, :N_NODES]], axis=-1)
    dn = (((1,), (1,)), ((), ()))
    t = lax.dot_general(x, w1_ref[...], dn,
                        preferred_element_type=jnp.float32,
                        precision=lax.Precision.DEFAULT) + b1_ref[...]
    t = _bn_relu(t, g1_ref[...], be1_ref[...])
    t = lax.dot_general(t, w2_ref[...], dn,
                        preferred_element_type=jnp.float32,
                        precision=lax.Precision.DEFAULT) + b2_ref[...]
    t = _bn_relu(t, g2_ref[...], be2_ref[...])
    t = _bn_relu(t, g3_ref[...], be3_ref[...])
    if split_out:
        out_ref[0, :N_NODES] = t[:, :H]
        out_ref[0, N_NODES:] = jnp.zeros((NP - N_NODES, H), jnp.float32)
        out_ref[1, :N_NODES] = t[:, H:]
        out_ref[1, N_NODES:] = jnp.zeros((NP - N_NODES, H), jnp.float32)
    else:
        out_ref[...] = t


def _mlp(x_split, w1, b1, w2, b2, g1, be1, g2, be2, g3, be3, split_out):
    out_shape = (jax.ShapeDtypeStruct((NC, NP, H), jnp.float32)
                 if split_out else
                 jax.ShapeDtypeStruct((N_NODES, D), jnp.float32))
    return pl.pallas_call(
        functools.partial(_mlp_body, split_out),
        out_shape=out_shape,
    )(x_split, w1, b1.reshape(1, D), w2, b2.reshape(1, D),
      g1.reshape(1, D), be1.reshape(1, D), g2.reshape(1, D),
      be2.reshape(1, D), g3.reshape(1, D), be3.reshape(1, D))


def kernel(h, edge_index, W1, b1, W2, b2, g1, be1, g2, be2, g3, be3):
    src = edge_index[0].astype(jnp.int32)
    dst = edge_index[1].astype(jnp.int32)
    x = jnp.pad(jnp.stack([h[:, :H], h[:, H:]]),
                ((0, 0), (0, NP - N_NODES), (0, 0)))
    for i in range(NUM_LAYERS):
        agg = _segment_sum(x, src, dst)
        x = _mlp(agg, W1[i], b1[i], W2[i], b2[i], g1[i], be1[i],
                 g2[i], be2[i], g3[i], be3[i],
                 split_out=(i < NUM_LAYERS - 1))
    return x
